# baseline (speedup 1.0000x reference)
import functools

import jax
import jax.numpy as jnp
from jax import lax
from jax.experimental import pallas as pl
from jax.experimental.pallas import tpu as pltpu
from jax.experimental.pallas import tpu_sc as plsc

_NS = 16
_NC = 2

_DNUMS = lax.GatherDimensionNumbers(
    offset_dims=(), collapsed_slice_dims=(0,), start_index_map=(0,))


def _make_deg_kernel(NP, CH, K):
  rows_per = NP // _NS
  zlen = ((rows_per + 15) // 16) * 16
  mesh = plsc.VectorSubcoreMesh(core_axis_name="c", subcore_axis_name="s")

  @functools.partial(
      pl.kernel,
      out_type=jax.ShapeDtypeStruct((_NC * NP,), jnp.float32),
      mesh=mesh,
      scratch_types=[
          pltpu.VMEM((CH, K), jnp.int32),
          pltpu.VMEM((CH, K), jnp.float32),
          pltpu.VMEM((zlen,), jnp.float32),
          pltpu.VMEM_SHARED((NP,), jnp.float32),
      ],
  )
  def deg_kernel(dsts, ews, out, dst_v, ew_v, zbuf, acc):
    cid = lax.axis_index("c")
    sid = lax.axis_index("s")
    wid = cid * _NS + sid
    pltpu.sync_copy(dsts.at[wid], dst_v)
    pltpu.sync_copy(ews.at[wid], ew_v)

    def zfill(i, c):
      zbuf[pl.ds(i * 16, 16)] = jnp.zeros((16,), jnp.float32)
      return c

    lax.fori_loop(0, zlen // 16, zfill, 0)
    pltpu.sync_copy(zbuf.at[pl.ds(0, rows_per)],
                    acc.at[pl.ds(sid * rows_per, rows_per)])
    plsc.subcore_barrier()

    def body(j, carry):
      pltpu.sync_copy(ew_v.at[j], acc.at[dst_v.at[j]], add=True)
      return carry

    lax.fori_loop(0, CH, body, 0)
    plsc.subcore_barrier()
    pltpu.sync_copy(acc.at[pl.ds(sid * rows_per, rows_per)],
                    zbuf.at[pl.ds(0, rows_per)])
    pltpu.sync_copy(zbuf.at[pl.ds(0, rows_per)],
                    out.at[pl.ds(cid * NP + sid * rows_per, rows_per)])

  return deg_kernel


def _make_agg_kernel(NP, CH, K, DH):
  rows_per = NP // _NS
  nv = DH // 16
  mesh = plsc.VectorSubcoreMesh(core_axis_name="c", subcore_axis_name="s")

  @functools.partial(
      pl.kernel,
      out_type=jax.ShapeDtypeStruct((_NC, NP, DH), jnp.float32),
      mesh=mesh,
      compiler_params=pltpu.CompilerParams(use_tc_tiling_on_sc=False),
      scratch_types=[
          pltpu.VMEM((CH, K), jnp.int32),
          pltpu.VMEM((CH, K), jnp.int32),
          pltpu.VMEM((K, DH), jnp.float32),
          pltpu.VMEM((K, DH), jnp.float32),
          pltpu.VMEM((K, DH), jnp.float32),
          pltpu.VMEM((K, DH), jnp.float32),
          pltpu.VMEM((K,), jnp.float32),
          pltpu.VMEM((K,), jnp.float32),
          pltpu.VMEM((K,), jnp.float32),
          pltpu.VMEM((K,), jnp.float32),
          [pltpu.SemaphoreType.DMA] * 4,
          [pltpu.SemaphoreType.DMA] * 4,
          [pltpu.SemaphoreType.DMA] * 4,
          pltpu.VMEM_SHARED((NP, DH), jnp.float32),
      ],
  )
  def agg_kernel(y, srcs, dsts, ews, out,
                 src_v, dst_v, rows0, rows1, rows2, rows3,
                 ew0, ew1, ew2, ew3, gsem, ssem, esem, acc):
    cid = lax.axis_index("c")
    sid = lax.axis_index("s")
    pltpu.sync_copy(srcs.at[sid], src_v)
    pltpu.sync_copy(dsts.at[sid], dst_v)
    ewh = ews.at[sid]

    def zrow(i, c):
      for v in range(nv):
        rows0[i, pl.ds(v * 16, 16)] = jnp.zeros((16,), jnp.float32)
      return c

    lax.fori_loop(0, K, zrow, 0)
    for t in range(-(-rows_per // K)):
      n = min(K, rows_per - t * K)
      pltpu.sync_copy(rows0.at[pl.ds(0, n)],
                      acc.at[pl.ds(sid * rows_per + t * K, n)])
    plsc.subcore_barrier()

    rows = (rows0, rows1, rows2, rows3)
    ewb = (ew0, ew1, ew2, ew3)
    yh = y.at[cid]
    H = K // 2

    def gstart(jj, slot):
      pltpu.async_copy(yh.at[src_v.at[jj, pl.ds(0, H)]],
                       rows[slot].at[pl.ds(0, H)], gsem[slot])
      pltpu.async_copy(yh.at[src_v.at[jj, pl.ds(H, H)]],
                       rows[slot].at[pl.ds(H, H)], gsem[slot])

    gstart(0, 0)
    gstart(1, 1)
    pltpu.async_copy(ewh.at[0], ew0, esem[0])
    pltpu.async_copy(ewh.at[1], ew1, esem[1])

    def outer(g, carry):
      for b in range(4):
        j = 4 * g + b
        rb = rows[b]
        eb = ewb[b]
        pltpu.make_async_copy(yh.at[src_v.at[0]], rb, gsem[b]).wait()
        pltpu.make_async_copy(ewh.at[0], eb, esem[b]).wait()

        def scale(g2, c):
          ev = eb[pl.ds(g2 * 16, 16)]
          for t in range(16):
            bv = lax.gather(ev, jnp.full((16, 1), t, jnp.int32), _DNUMS, (1,),
                            mode=lax.GatherScatterMode.PROMISE_IN_BOUNDS)
            i = g2 * 16 + t
            for v in range(nv):
              sl = pl.ds(v * 16, 16)
              rb[i, sl] = rb[i, sl] * bv
          return c

        lax.fori_loop(0, K // 16, scale, 0, unroll=4)
        pltpu.async_copy(rb, acc.at[dst_v.at[j]], ssem[b], add=True)

        b2 = (b + 2) % 4

        @pl.when(j >= 2)
        def _():
          pltpu.make_async_copy(rows[b2], acc.at[dst_v.at[0]], ssem[b2]).wait()

        @pl.when(j + 2 < CH)
        def _():
          gstart(j + 2, b2)
          pltpu.async_copy(ewh.at[j + 2], ewb[b2], esem[b2])

      return carry

    lax.fori_loop(0, CH // 4, outer, 0)
    for b in ((CH - 2) % 4, (CH - 1) % 4):
      pltpu.make_async_copy(rows[b], acc.at[dst_v.at[0]], ssem[b]).wait()
    plsc.subcore_barrier()
    for t in range(-(-rows_per // K)):
      n = min(K, rows_per - t * K)
      base = sid * rows_per + t * K
      pltpu.sync_copy(acc.at[pl.ds(base, n)], rows0.at[pl.ds(0, n)])
      pltpu.sync_copy(rows0.at[pl.ds(0, n)], out.at[cid, pl.ds(base, n)])

  return agg_kernel


_R = 1000


def _dense_first(x, W, degA, degB):
  N, Din = x.shape
  D = W.shape[1]
  DH = D // 2

  def body(x_ref, w_ref, da_ref, db_ref, y_ref, dinv_ref):
    deg = da_ref[...] + db_ref[...] + 1.0
    dinv = lax.rsqrt(deg)
    xw = jnp.dot(x_ref[...], w_ref[...], preferred_element_type=jnp.float32)
    y = xw * dinv
    y_ref[0] = y[:, :DH]
    y_ref[1] = y[:, DH:]
    dinv_ref[...] = dinv

  return pl.pallas_call(
      body,
      grid=(N // _R,),
      in_specs=[
          pl.BlockSpec((_R, Din), lambda i: (i, 0)),
          pl.BlockSpec((Din, D), lambda i: (0, 0)),
          pl.BlockSpec((_R, 1), lambda i: (i, 0)),
          pl.BlockSpec((_R, 1), lambda i: (i, 0)),
      ],
      out_specs=[
          pl.BlockSpec((2, _R, DH), lambda i: (0, i, 0)),
          pl.BlockSpec((_R, 1), lambda i: (i, 0)),
      ],
      out_shape=[
          jax.ShapeDtypeStruct((2, N, DH), jnp.float32),
          jax.ShapeDtypeStruct((N, 1), jnp.float32),
      ],
  )(x, W, degA, degB)


def _dense_mid(aggs, ys, dinv, b_prev, W_next):
  _, N, DH = ys.shape
  D = 2 * DH

  def body(a_ref, y_ref, dinv_ref, b_ref, w_ref, out_ref):
    dinv = dinv_ref[...]
    t0 = a_ref[0] + y_ref[0]
    t1 = a_ref[1] + y_ref[1]
    tf = jnp.concatenate([t0, t1], axis=1)
    h = jnp.maximum(dinv * tf + b_ref[...], 0.0)
    hw = jnp.dot(h, w_ref[...], preferred_element_type=jnp.float32)
    y = hw * dinv
    out_ref[0] = y[:, :DH]
    out_ref[1] = y[:, DH:]

  return pl.pallas_call(
      body,
      grid=(N // _R,),
      in_specs=[
          pl.BlockSpec((2, _R, DH), lambda i: (0, i, 0)),
          pl.BlockSpec((2, _R, DH), lambda i: (0, i, 0)),
          pl.BlockSpec((_R, 1), lambda i: (i, 0)),
          pl.BlockSpec((1, D), lambda i: (0, 0)),
          pl.BlockSpec((D, D), lambda i: (0, 0)),
      ],
      out_specs=pl.BlockSpec((2, _R, DH), lambda i: (0, i, 0)),
      out_shape=jax.ShapeDtypeStruct((2, N, DH), jnp.float32),
  )(aggs, ys, dinv, b_prev, W_next)


def _dense_final(aggs, ys, dinv, b3):
  _, N, DH = ys.shape
  D = 2 * DH

  def body(a_ref, y_ref, dinv_ref, b_ref, out_ref):
    t0 = a_ref[0] + y_ref[0]
    t1 = a_ref[1] + y_ref[1]
    tf = jnp.concatenate([t0, t1], axis=1)
    z = dinv_ref[...] * tf + b_ref[...]
    m = jnp.max(z, axis=1, keepdims=True)
    zs = z - m
    lse = jnp.log(jnp.sum(jnp.exp(zs), axis=1, keepdims=True))
    out_ref[...] = zs - lse

  return pl.pallas_call(
      body,
      grid=(N // _R,),
      in_specs=[
          pl.BlockSpec((2, _R, DH), lambda i: (0, i, 0)),
          pl.BlockSpec((2, _R, DH), lambda i: (0, i, 0)),
          pl.BlockSpec((_R, 1), lambda i: (i, 0)),
          pl.BlockSpec((1, D), lambda i: (0, 0)),
      ],
      out_specs=pl.BlockSpec((_R, D), lambda i: (i, 0)),
      out_shape=jax.ShapeDtypeStruct((N, D), jnp.float32),
  )(aggs, ys, dinv, b3)


def kernel(x, edge_index, edge_weight, W1, b1, W2, b2, W3, b3):
  N, _ = x.shape
  D = W1.shape[1]
  DH = D // 2
  E = edge_index.shape[1]

  src = edge_index[0].astype(jnp.int32)
  dst = edge_index[1].astype(jnp.int32)
  ew = edge_weight.astype(jnp.float32)

  K = 128
  CHA = -(-E // (_NS * K))
  CHA += (-CHA) % 4
  EP = _NS * CHA * K
  CHD = CHA // 2
  padn = EP - E
  pad_idx = jnp.arange(padn, dtype=jnp.int32) % N
  src_p = jnp.concatenate([src, pad_idx])
  dst_p = jnp.concatenate([dst, pad_idx])
  ew_p = jnp.concatenate([ew, jnp.zeros((padn,), jnp.float32)])
  dsts32 = dst_p.reshape(_NC * _NS, CHD, K)
  ews32 = ew_p.reshape(_NC * _NS, CHD, K)
  srcs16 = src_p.reshape(_NS, CHA, K)
  dsts16 = dst_p.reshape(_NS, CHA, K)
  ews16 = ew_p.reshape(_NS, CHA, K)

  NP = -(-N // 128) * 128

  degs = _make_deg_kernel(NP, CHD, K)(dsts32, ews32)
  degA = degs[:N, None]
  degB = degs[NP:NP + N, None]

  y1, dinv = _dense_first(x, W1, degA, degB)

  agg = _make_agg_kernel(NP, CHA, K, DH)
  acc1 = agg(y1, srcs16, dsts16, ews16)[:, :N]
  y2 = _dense_mid(acc1, y1, dinv, b1.reshape(1, D), W2)
  acc2 = agg(y2, srcs16, dsts16, ews16)[:, :N]
  y3 = _dense_mid(acc2, y2, dinv, b2.reshape(1, D), W3)
  acc3 = agg(y3, srcs16, dsts16, ews16)[:, :N]
  return _dense_final(acc3, y3, dinv, b3.reshape(1, D))


# speedup vs baseline: 1.0077x; 1.0077x over previous
import functools

import jax
import jax.numpy as jnp
from jax import lax
from jax.experimental import pallas as pl
from jax.experimental.pallas import tpu as pltpu
from jax.experimental.pallas import tpu_sc as plsc

_NS = 16
_NC = 2

_DNUMS = lax.GatherDimensionNumbers(
    offset_dims=(), collapsed_slice_dims=(0,), start_index_map=(0,))


def _make_deg_kernel(NP, CH, K):
  rows_per = NP // _NS
  zlen = ((rows_per + 15) // 16) * 16
  mesh = plsc.VectorSubcoreMesh(core_axis_name="c", subcore_axis_name="s")

  @functools.partial(
      pl.kernel,
      out_type=jax.ShapeDtypeStruct((_NC * NP,), jnp.float32),
      mesh=mesh,
      scratch_types=[
          pltpu.VMEM((CH, K), jnp.int32),
          pltpu.VMEM((CH, K), jnp.float32),
          pltpu.VMEM((zlen,), jnp.float32),
          pltpu.VMEM_SHARED((NP,), jnp.float32),
      ],
  )
  def deg_kernel(dsts, ews, out, dst_v, ew_v, zbuf, acc):
    cid = lax.axis_index("c")
    sid = lax.axis_index("s")
    wid = cid * _NS + sid
    pltpu.sync_copy(dsts.at[wid], dst_v)
    pltpu.sync_copy(ews.at[wid], ew_v)

    def zfill(i, c):
      zbuf[pl.ds(i * 16, 16)] = jnp.zeros((16,), jnp.float32)
      return c

    lax.fori_loop(0, zlen // 16, zfill, 0)
    pltpu.sync_copy(zbuf.at[pl.ds(0, rows_per)],
                    acc.at[pl.ds(sid * rows_per, rows_per)])
    plsc.subcore_barrier()

    def body(j, carry):
      pltpu.sync_copy(ew_v.at[j], acc.at[dst_v.at[j]], add=True)
      return carry

    lax.fori_loop(0, CH, body, 0)
    plsc.subcore_barrier()
    pltpu.sync_copy(acc.at[pl.ds(sid * rows_per, rows_per)],
                    zbuf.at[pl.ds(0, rows_per)])
    pltpu.sync_copy(zbuf.at[pl.ds(0, rows_per)],
                    out.at[pl.ds(cid * NP + sid * rows_per, rows_per)])

  return deg_kernel


def _make_agg_kernel(NP, CH, K, DH):
  rows_per = NP // _NS
  nv = DH // 16
  mesh = plsc.VectorSubcoreMesh(core_axis_name="c", subcore_axis_name="s")

  @functools.partial(
      pl.kernel,
      out_type=jax.ShapeDtypeStruct((_NC, NP, DH), jnp.float32),
      mesh=mesh,
      compiler_params=pltpu.CompilerParams(use_tc_tiling_on_sc=False),
      scratch_types=[
          pltpu.VMEM((CH, K), jnp.int32),
          pltpu.VMEM((CH, K), jnp.int32),
          pltpu.VMEM((K, DH), jnp.float32),
          pltpu.VMEM((K, DH), jnp.float32),
          pltpu.VMEM((K, DH), jnp.float32),
          pltpu.VMEM((K, DH), jnp.float32),
          pltpu.VMEM((K,), jnp.float32),
          pltpu.VMEM((K,), jnp.float32),
          pltpu.VMEM((K,), jnp.float32),
          pltpu.VMEM((K,), jnp.float32),
          [pltpu.SemaphoreType.DMA] * 4,
          [pltpu.SemaphoreType.DMA] * 4,
          [pltpu.SemaphoreType.DMA] * 4,
          pltpu.VMEM_SHARED((NP, DH), jnp.float32),
      ],
  )
  def agg_kernel(y, srcs, dsts, ews, out,
                 src_v, dst_v, rows0, rows1, rows2, rows3,
                 ew0, ew1, ew2, ew3, gsem, ssem, esem, acc):
    cid = lax.axis_index("c")
    sid = lax.axis_index("s")
    pltpu.sync_copy(srcs.at[sid], src_v)
    pltpu.sync_copy(dsts.at[sid], dst_v)
    ewh = ews.at[sid]

    def zrow(i, c):
      for v in range(nv):
        rows0[i, pl.ds(v * 16, 16)] = jnp.zeros((16,), jnp.float32)
      return c

    lax.fori_loop(0, K, zrow, 0)
    for t in range(-(-rows_per // K)):
      n = min(K, rows_per - t * K)
      pltpu.sync_copy(rows0.at[pl.ds(0, n)],
                      acc.at[pl.ds(sid * rows_per + t * K, n)])
    plsc.subcore_barrier()

    rows = (rows0, rows1, rows2, rows3)
    ewb = (ew0, ew1, ew2, ew3)
    yh = y.at[cid]
    H = K // 2

    def gstart(jj, slot):
      pltpu.async_copy(yh.at[src_v.at[jj, pl.ds(0, H)]],
                       rows[slot].at[pl.ds(0, H)], gsem[slot])
      pltpu.async_copy(yh.at[src_v.at[jj, pl.ds(H, H)]],
                       rows[slot].at[pl.ds(H, H)], gsem[slot])

    gstart(0, 0)
    gstart(1, 1)
    pltpu.async_copy(ewh.at[0], ew0, esem[0])
    pltpu.async_copy(ewh.at[1], ew1, esem[1])

    def outer(g, carry):
      for b in range(4):
        j = 4 * g + b
        rb = rows[b]
        eb = ewb[b]
        pltpu.make_async_copy(yh.at[src_v.at[0]], rb, gsem[b]).wait()
        pltpu.make_async_copy(ewh.at[0], eb, esem[b]).wait()

        def scale(g2, c):
          ev = eb[pl.ds(g2 * 16, 16)]
          for t in range(16):
            bv = lax.gather(ev, jnp.full((16, 1), t, jnp.int32), _DNUMS, (1,),
                            mode=lax.GatherScatterMode.PROMISE_IN_BOUNDS)
            i = g2 * 16 + t
            for v in range(nv):
              sl = pl.ds(v * 16, 16)
              rb[i, sl] = rb[i, sl] * bv
          return c

        lax.fori_loop(0, K // 16, scale, 0, unroll=2)
        pltpu.async_copy(rb, acc.at[dst_v.at[j]], ssem[b], add=True)

        b2 = (b + 2) % 4

        @pl.when(j >= 2)
        def _():
          pltpu.make_async_copy(rows[b2], acc.at[dst_v.at[0]], ssem[b2]).wait()

        @pl.when(j + 2 < CH)
        def _():
          gstart(j + 2, b2)
          pltpu.async_copy(ewh.at[j + 2], ewb[b2], esem[b2])

      return carry

    lax.fori_loop(0, CH // 4, outer, 0)
    for b in ((CH - 2) % 4, (CH - 1) % 4):
      pltpu.make_async_copy(rows[b], acc.at[dst_v.at[0]], ssem[b]).wait()
    plsc.subcore_barrier()
    for t in range(-(-rows_per // K)):
      n = min(K, rows_per - t * K)
      base = sid * rows_per + t * K
      pltpu.sync_copy(acc.at[pl.ds(base, n)], rows0.at[pl.ds(0, n)])
      pltpu.sync_copy(rows0.at[pl.ds(0, n)], out.at[cid, pl.ds(base, n)])

  return agg_kernel


_R = 1000


def _dense_first(x, W, degA, degB):
  N, Din = x.shape
  D = W.shape[1]
  DH = D // 2

  def body(x_ref, w_ref, da_ref, db_ref, y_ref, dinv_ref):
    deg = da_ref[...] + db_ref[...] + 1.0
    dinv = lax.rsqrt(deg)
    xw = jnp.dot(x_ref[...], w_ref[...], preferred_element_type=jnp.float32)
    y = xw * dinv
    y_ref[0] = y[:, :DH]
    y_ref[1] = y[:, DH:]
    dinv_ref[...] = dinv

  return pl.pallas_call(
      body,
      grid=(N // _R,),
      in_specs=[
          pl.BlockSpec((_R, Din), lambda i: (i, 0)),
          pl.BlockSpec((Din, D), lambda i: (0, 0)),
          pl.BlockSpec((_R, 1), lambda i: (i, 0)),
          pl.BlockSpec((_R, 1), lambda i: (i, 0)),
      ],
      out_specs=[
          pl.BlockSpec((2, _R, DH), lambda i: (0, i, 0)),
          pl.BlockSpec((_R, 1), lambda i: (i, 0)),
      ],
      out_shape=[
          jax.ShapeDtypeStruct((2, N, DH), jnp.float32),
          jax.ShapeDtypeStruct((N, 1), jnp.float32),
      ],
  )(x, W, degA, degB)


def _dense_mid(aggs, ys, dinv, b_prev, W_next):
  _, N, DH = ys.shape
  D = 2 * DH

  def body(a_ref, y_ref, dinv_ref, b_ref, w_ref, out_ref):
    dinv = dinv_ref[...]
    t0 = a_ref[0] + y_ref[0]
    t1 = a_ref[1] + y_ref[1]
    tf = jnp.concatenate([t0, t1], axis=1)
    h = jnp.maximum(dinv * tf + b_ref[...], 0.0)
    hw = jnp.dot(h, w_ref[...], preferred_element_type=jnp.float32)
    y = hw * dinv
    out_ref[0] = y[:, :DH]
    out_ref[1] = y[:, DH:]

  return pl.pallas_call(
      body,
      grid=(N // _R,),
      in_specs=[
          pl.BlockSpec((2, _R, DH), lambda i: (0, i, 0)),
          pl.BlockSpec((2, _R, DH), lambda i: (0, i, 0)),
          pl.BlockSpec((_R, 1), lambda i: (i, 0)),
          pl.BlockSpec((1, D), lambda i: (0, 0)),
          pl.BlockSpec((D, D), lambda i: (0, 0)),
      ],
      out_specs=pl.BlockSpec((2, _R, DH), lambda i: (0, i, 0)),
      out_shape=jax.ShapeDtypeStruct((2, N, DH), jnp.float32),
  )(aggs, ys, dinv, b_prev, W_next)


def _dense_final(aggs, ys, dinv, b3):
  _, N, DH = ys.shape
  D = 2 * DH

  def body(a_ref, y_ref, dinv_ref, b_ref, out_ref):
    t0 = a_ref[0] + y_ref[0]
    t1 = a_ref[1] + y_ref[1]
    tf = jnp.concatenate([t0, t1], axis=1)
    z = dinv_ref[...] * tf + b_ref[...]
    m = jnp.max(z, axis=1, keepdims=True)
    zs = z - m
    lse = jnp.log(jnp.sum(jnp.exp(zs), axis=1, keepdims=True))
    out_ref[...] = zs - lse

  return pl.pallas_call(
      body,
      grid=(N // _R,),
      in_specs=[
          pl.BlockSpec((2, _R, DH), lambda i: (0, i, 0)),
          pl.BlockSpec((2, _R, DH), lambda i: (0, i, 0)),
          pl.BlockSpec((_R, 1), lambda i: (i, 0)),
          pl.BlockSpec((1, D), lambda i: (0, 0)),
      ],
      out_specs=pl.BlockSpec((_R, D), lambda i: (i, 0)),
      out_shape=jax.ShapeDtypeStruct((N, D), jnp.float32),
  )(aggs, ys, dinv, b3)


def kernel(x, edge_index, edge_weight, W1, b1, W2, b2, W3, b3):
  N, _ = x.shape
  D = W1.shape[1]
  DH = D // 2
  E = edge_index.shape[1]

  src = edge_index[0].astype(jnp.int32)
  dst = edge_index[1].astype(jnp.int32)
  ew = edge_weight.astype(jnp.float32)

  K = 128
  CHA = -(-E // (_NS * K))
  CHA += (-CHA) % 4
  EP = _NS * CHA * K
  CHD = CHA // 2
  padn = EP - E
  pad_idx = jnp.arange(padn, dtype=jnp.int32) % N
  src_p = jnp.concatenate([src, pad_idx])
  dst_p = jnp.concatenate([dst, pad_idx])
  ew_p = jnp.concatenate([ew, jnp.zeros((padn,), jnp.float32)])
  dsts32 = dst_p.reshape(_NC * _NS, CHD, K)
  ews32 = ew_p.reshape(_NC * _NS, CHD, K)
  srcs16 = src_p.reshape(_NS, CHA, K)
  dsts16 = dst_p.reshape(_NS, CHA, K)
  ews16 = ew_p.reshape(_NS, CHA, K)

  NP = -(-N // 128) * 128

  degs = _make_deg_kernel(NP, CHD, K)(dsts32, ews32)
  degA = degs[:N, None]
  degB = degs[NP:NP + N, None]

  y1, dinv = _dense_first(x, W1, degA, degB)

  agg = _make_agg_kernel(NP, CHA, K, DH)
  acc1 = agg(y1, srcs16, dsts16, ews16)[:, :N]
  y2 = _dense_mid(acc1, y1, dinv, b1.reshape(1, D), W2)
  acc2 = agg(y2, srcs16, dsts16, ews16)[:, :N]
  y3 = _dense_mid(acc2, y2, dinv, b2.reshape(1, D), W3)
  acc3 = agg(y3, srcs16, dsts16, ews16)[:, :N]
  return _dense_final(acc3, y3, dinv, b3.reshape(1, D))
